# baseline (speedup 1.0000x reference)
import functools

import jax
import jax.numpy as jnp
from jax import lax
from jax.experimental import pallas as pl
from jax.experimental.pallas import tpu as pltpu
from jax.experimental.pallas import tpu_sc as plsc

N = 10000
E = 160000
D = 128
H = 128

NC = 2
NS = 16
NW = NC * NS

BLK = 128
NP = 10112
NB = NP // BLK
RPT = NP // NS
EP = 163840
ER = EP // BLK
ERW = ER // NW


def _mesh():
    return plsc.VectorSubcoreMesh(
        core_axis_name="c", subcore_axis_name="s", num_cores=NC, num_subcores=NS)


DP = 10240


def _deg_body(dst_hbm, out_hbm, dst_v, degp):
    c = lax.axis_index("c")
    s = lax.axis_index("s")
    wid = s * NC + c
    zer = jnp.zeros((16,), jnp.float32)
    one = jnp.full((16,), 1.0, jnp.float32)

    def zfill(i, carry):
        degp[pl.ds(i * 16, 16)] = zer
        return carry
    lax.fori_loop(0, DP // 16, zfill, 0)
    pltpu.sync_copy(dst_hbm.at[pl.ds(wid * ERW, ERW), :], dst_v)

    def body(i, carry):
        j = i // 8
        k = (i % 8) * 16
        d = dst_v[j, pl.ds(k, 16)]
        plsc.addupdate_scatter(degp, [d], one)
        return carry
    lax.fori_loop(0, ERW * 8, body, 0)
    pltpu.sync_copy(degp, out_hbm.at[pl.ds(wid * DP, DP)])


def _deg_call(dst2d):
    fn = pl.kernel(
        _deg_body,
        out_type=jax.ShapeDtypeStruct((NW * DP,), jnp.float32),
        mesh=_mesh(),
        compiler_params=pltpu.CompilerParams(needs_layout_passes=False),
        scratch_types=[
            pltpu.VMEM((ERW, BLK), jnp.int32),
            pltpu.VMEM((DP,), jnp.float32),
        ],
    )
    return fn(dst2d)


KT = ER // NS


def _agg_body(hs_hbm, src_hbm, dst_hbm, out_hbm, src_v, dst_r, buf_a, buf_b,
              agg_sh, sem, sem_i):
    c = lax.axis_index("c")
    s = lax.axis_index("s")
    zer = jnp.zeros((16,), jnp.float32)

    def zfill(i, carry):
        r = i // 8
        k = (i % 8) * 16
        buf_a[r, pl.ds(k, 16)] = zer
        return carry
    lax.fori_loop(0, BLK * 8, zfill, 0)

    r0 = s * RPT
    for k in range(RPT // BLK):
        pltpu.sync_copy(buf_a, agg_sh.at[pl.ds(r0 + k * BLK, BLK), :])
    rem = RPT % BLK
    if rem:
        pltpu.sync_copy(buf_a.at[pl.ds(0, rem), :],
                        agg_sh.at[pl.ds(r0 + (RPT // BLK) * BLK, rem), :])

    @pl.when(c == 0)
    def _():
        base = s * KT
        NBLK = KT // 8
        pltpu.sync_copy(src_hbm.at[pl.ds(base, KT), :], src_v)
        pltpu.async_copy(dst_hbm.at[pl.ds(base, 8), :],
                         dst_r.at[pl.ds(0, 8), :], sem_i)
        pltpu.async_copy(dst_hbm.at[pl.ds(base + 8, 8), :],
                         dst_r.at[pl.ds(8, 8), :], sem_i)
        pltpu.async_copy(hs_hbm.at[src_v.at[0]], buf_a, sem)

        def body(b, carry):
            slot = (b % 2) * 8
            pltpu.make_async_copy(dst_hbm.at[pl.ds(base + b * 8, 8), :],
                                  dst_r.at[pl.ds(slot, 8), :], sem_i).wait()
            for u in range(8):
                j = b * 8 + u
                buf = buf_a if u % 2 == 0 else buf_b
                nbuf = buf_b if u % 2 == 0 else buf_a
                pltpu.make_async_copy(hs_hbm.at[src_v.at[j]], buf, sem).wait()

                @pl.when(j + 1 < KT)
                def _():
                    pltpu.async_copy(hs_hbm.at[src_v.at[j + 1]], nbuf, sem)
                pltpu.sync_copy(buf, agg_sh.at[dst_r.at[slot + u]], add=True)

            @pl.when(b + 2 < NBLK)
            def _():
                pltpu.async_copy(dst_hbm.at[pl.ds(base + (b + 2) * 8, 8), :],
                                 dst_r.at[pl.ds(slot, 8), :], sem_i)
            return carry
        lax.fori_loop(0, NBLK, body, 0)
    plsc.subcore_barrier()

    @pl.when(c == 0)
    def _():
        for k in range(RPT // BLK):
            pltpu.sync_copy(agg_sh.at[pl.ds(r0 + k * BLK, BLK), :],
                            out_hbm.at[pl.ds(r0 + k * BLK, BLK), :])
        if rem:
            pltpu.sync_copy(agg_sh.at[pl.ds(r0 + (RPT // BLK) * BLK, rem), :],
                            out_hbm.at[pl.ds(r0 + (RPT // BLK) * BLK, rem), :])


def _agg_call(hs, src2d, dst2d):
    fn = pl.kernel(
        _agg_body,
        out_type=jax.ShapeDtypeStruct((NP, D), jnp.float32),
        mesh=_mesh(),
        scratch_types=[
            pltpu.VMEM((KT, BLK), jnp.int32),
            pltpu.VMEM((16, BLK), jnp.int32),
            pltpu.VMEM((BLK, D), jnp.float32),
            pltpu.VMEM((BLK, D), jnp.float32),
            pltpu.VMEM_SHARED((NP, D), jnp.float32),
            pltpu.SemaphoreType.DMA,
            pltpu.SemaphoreType.DMA,
        ],
    )
    return fn(hs, src2d, dst2d)


def _hs_body(x_ref, deg_ref, w_ref, hs_ref, dinv_ref):
    i = pl.program_id(0)
    deg_row = jnp.sum(deg_ref[...], axis=0, keepdims=True) + 1.0
    eye = (lax.broadcasted_iota(jnp.int32, (BLK, BLK), 0)
           == lax.broadcasted_iota(jnp.int32, (BLK, BLK), 1)
           ).astype(jnp.float32)
    dn = (((1,), (1,)), ((), ()))
    deg_col = lax.dot_general(eye, deg_row, dn,
                              preferred_element_type=jnp.float32)
    dinv = lax.rsqrt(deg_col)
    rows = i * BLK + lax.broadcasted_iota(jnp.int32, (BLK, 1), 0)
    valid = rows < N
    dinv = jnp.where(valid, dinv, 1.0)
    h = jnp.dot(x_ref[...], w_ref[...], preferred_element_type=jnp.float32)
    hs = jnp.where(valid, h * dinv, 0.0)
    hs_ref[...] = hs
    dinv_ref[...] = dinv


def _hs_call(x, deg_all, w_conv2):
    return pl.pallas_call(
        _hs_body,
        grid=(NB,),
        in_specs=[
            pl.BlockSpec((BLK, D), lambda i: (i, 0)),
            pl.BlockSpec((NW, BLK), lambda i: (0, i)),
            pl.BlockSpec((D, D), lambda i: (0, 0)),
        ],
        out_specs=[
            pl.BlockSpec((BLK, D), lambda i: (i, 0)),
            pl.BlockSpec((BLK, 1), lambda i: (i, 0)),
        ],
        out_shape=[
            jax.ShapeDtypeStruct((NP, D), jnp.float32),
            jax.ShapeDtypeStruct((NP, 1), jnp.float32),
        ],
    )(x, deg_all, w_conv2)


def _ne_body(agg_ref, hs_ref, dinv_ref, bc_ref, wl_ref, bl_ref,
             ne_ref, gs_ref):
    i = pl.program_id(0)
    agg = agg_ref[...] + hs_ref[...]
    conv = jnp.maximum(agg * dinv_ref[...] + bc_ref[...], 0.0)
    ne = jnp.dot(conv, wl_ref[...], preferred_element_type=jnp.float32)
    ne = jnp.maximum(ne + bl_ref[...], 0.0)
    rows = i * BLK + lax.broadcasted_iota(jnp.int32, (BLK, 1), 0)
    ne = jnp.where(rows < N, ne, 0.0)
    ne_ref[...] = ne

    @pl.when(i == 0)
    def _():
        gs_ref[...] = jnp.zeros_like(gs_ref)
    gs_ref[...] += jnp.sum(ne, axis=0, keepdims=True)


def _ne_call(agg, hs, dinv, b_conv2, w_lin1, b_lin1):
    return pl.pallas_call(
        _ne_body,
        grid=(NB,),
        in_specs=[
            pl.BlockSpec((BLK, D), lambda i: (i, 0)),
            pl.BlockSpec((BLK, D), lambda i: (i, 0)),
            pl.BlockSpec((BLK, 1), lambda i: (i, 0)),
            pl.BlockSpec((1, D), lambda i: (0, 0)),
            pl.BlockSpec((D, D), lambda i: (0, 0)),
            pl.BlockSpec((1, D), lambda i: (0, 0)),
        ],
        out_specs=[
            pl.BlockSpec((BLK, D), lambda i: (i, 0)),
            pl.BlockSpec((1, D), lambda i: (0, 0)),
        ],
        out_shape=[
            jax.ShapeDtypeStruct((NP, D), jnp.float32),
            jax.ShapeDtypeStruct((1, D), jnp.float32),
        ],
    )(agg, hs, dinv, b_conv2.reshape(1, D), w_lin1,
      b_lin1.reshape(1, D))


def _q_body(ne_ref, t_ref, gs_ref, wm_ref, bm_ref, wo_ref, bo_ref, q_ref):
    g = gs_ref[...] * (1.0 / N)
    wm = wm_ref[...]
    bias = jnp.dot(g, wm[D:, :], preferred_element_type=jnp.float32)
    bias = bias + bm_ref[...]
    hid = jnp.dot(ne_ref[...], wm[:D, :], preferred_element_type=jnp.float32)
    hid = jnp.maximum(hid + bias, 0.0)
    t = t_ref[...]
    dn = (((1,), (1,)), ((), ()))
    v = lax.dot_general(wo_ref[...], t, dn,
                        preferred_element_type=jnp.float32)
    c = lax.dot_general(bo_ref[...], t, dn,
                        preferred_element_type=jnp.float32)
    q_ref[...] = jnp.dot(hid, v, preferred_element_type=jnp.float32) + c


def _q_call(ne, t_row, gsum, w_mlp, b_mlp, w_out, b_out):
    return pl.pallas_call(
        _q_body,
        grid=(NB,),
        in_specs=[
            pl.BlockSpec((BLK, D), lambda i: (i, 0)),
            pl.BlockSpec((1, D), lambda i: (0, 0)),
            pl.BlockSpec((1, D), lambda i: (0, 0)),
            pl.BlockSpec((2 * D, H), lambda i: (0, 0)),
            pl.BlockSpec((1, H), lambda i: (0, 0)),
            pl.BlockSpec((H, D), lambda i: (0, 0)),
            pl.BlockSpec((1, D), lambda i: (0, 0)),
        ],
        out_specs=pl.BlockSpec((BLK, 1), lambda i: (i, 0)),
        out_shape=jax.ShapeDtypeStruct((N, 1), jnp.float32),
    )(ne, t_row, gsum, w_mlp, b_mlp.reshape(1, H), w_out,
      b_out.reshape(1, D))


def kernel(x, edge_index, target_node, W_conv2, b_conv2, W_lin1, b_lin1,
           W_mlp, b_mlp, W_out, b_out):
    src2d = jnp.pad(edge_index[0], (0, EP - E),
                    constant_values=N).reshape(ER, BLK)
    dst2d = jnp.pad(edge_index[1], (0, EP - E),
                    constant_values=N).reshape(ER, BLK)

    degr = _deg_call(dst2d)
    deg_all = degr.reshape(NW, DP)
    hs, dinv = _hs_call(x, deg_all, W_conv2)
    agg = _agg_call(hs, src2d, dst2d)
    ne, gsum = _ne_call(agg, hs, dinv, b_conv2, W_lin1, b_lin1)
    t_row = lax.dynamic_slice(ne, (target_node, 0), (1, D))
    return _q_call(ne, t_row, gsum, W_mlp, b_mlp, W_out, b_out)


# speedup vs baseline: 1.1793x; 1.1793x over previous
import functools

import jax
import jax.numpy as jnp
from jax import lax
from jax.experimental import pallas as pl
from jax.experimental.pallas import tpu as pltpu
from jax.experimental.pallas import tpu_sc as plsc

N = 10000
E = 160000
D = 128
H = 128

NC = 2
NS = 16
NW = NC * NS

BLK = 128
NP = 10112
NB = NP // BLK
RPT = NP // NS
EP = 163840
ER = EP // BLK
ERW = ER // NW


def _mesh():
    return plsc.VectorSubcoreMesh(
        core_axis_name="c", subcore_axis_name="s", num_cores=NC, num_subcores=NS)


DP = 10240


def _deg_body(dst_hbm, out_hbm, dst_v, degp):
    c = lax.axis_index("c")
    s = lax.axis_index("s")
    wid = s * NC + c
    zer = jnp.zeros((16,), jnp.float32)
    one = jnp.full((16,), 1.0, jnp.float32)

    def zfill(i, carry):
        degp[pl.ds(i * 16, 16)] = zer
        return carry
    lax.fori_loop(0, DP // 16, zfill, 0)
    pltpu.sync_copy(dst_hbm.at[pl.ds(wid * ERW, ERW), :], dst_v)

    def body(i, carry):
        j = i // 8
        k = (i % 8) * 16
        d = dst_v[j, pl.ds(k, 16)]
        plsc.addupdate_scatter(degp, [d], one)
        return carry
    lax.fori_loop(0, ERW * 8, body, 0)
    pltpu.sync_copy(degp, out_hbm.at[pl.ds(wid * DP, DP)])


def _deg_call(dst2d):
    fn = pl.kernel(
        _deg_body,
        out_type=jax.ShapeDtypeStruct((NW * DP,), jnp.float32),
        mesh=_mesh(),
        compiler_params=pltpu.CompilerParams(needs_layout_passes=False),
        scratch_types=[
            pltpu.VMEM((ERW, BLK), jnp.int32),
            pltpu.VMEM((DP,), jnp.float32),
        ],
    )
    return fn(dst2d)


K0 = 64
K1 = ER // NS - K0


def _agg_body(hs_hbm, src_hbm, dst_hbm, out_hbm, src_v, dst_r, buf_a, buf_b,
              agg_sh, sem, sem_i):
    c = lax.axis_index("c")
    s = lax.axis_index("s")
    zer = jnp.zeros((16,), jnp.float32)

    def zfill(i, carry):
        r = i // 8
        k = (i % 8) * 16
        buf_a[r, pl.ds(k, 16)] = zer
        return carry
    lax.fori_loop(0, BLK * 8, zfill, 0)

    r0 = s * RPT
    for k in range(RPT // BLK):
        pltpu.sync_copy(buf_a, agg_sh.at[pl.ds(r0 + k * BLK, BLK), :])
    rem = RPT % BLK
    if rem:
        pltpu.sync_copy(buf_a.at[pl.ds(0, rem), :],
                        agg_sh.at[pl.ds(r0 + (RPT // BLK) * BLK, rem), :])

    def run_edges(K, base):
        pltpu.sync_copy(src_hbm.at[pl.ds(base, K), :], src_v.at[pl.ds(0, K), :])
        pltpu.sync_copy(dst_hbm.at[pl.ds(base, K), :], dst_r.at[pl.ds(0, K), :])
        pltpu.async_copy(hs_hbm.at[src_v.at[0]], buf_a, sem)

        def body(t, carry):
            j0 = 2 * t
            j1 = j0 + 1
            pltpu.make_async_copy(hs_hbm.at[src_v.at[j0]], buf_a, sem).wait()
            pltpu.async_copy(hs_hbm.at[src_v.at[j1]], buf_b, sem)
            pltpu.sync_copy(buf_a, agg_sh.at[dst_r.at[j0]], add=True)
            pltpu.make_async_copy(hs_hbm.at[src_v.at[j1]], buf_b, sem).wait()

            @pl.when(j1 + 1 < K)
            def _():
                pltpu.async_copy(hs_hbm.at[src_v.at[j1 + 1]], buf_a, sem)
            pltpu.sync_copy(buf_b, agg_sh.at[dst_r.at[j1]], add=True)
            return carry
        lax.fori_loop(0, K // 2, body, 0)

    @pl.when(c == 0)
    def _():
        run_edges(K0, s * K0)

    if K1 > 0:
        @pl.when(c == 1)
        def _():
            run_edges(K1, NS * K0 + s * K1)
    plsc.subcore_barrier()
    for k in range(RPT // BLK):
        pltpu.sync_copy(agg_sh.at[pl.ds(r0 + k * BLK, BLK), :],
                        out_hbm.at[pl.ds(c * NP + r0 + k * BLK, BLK), :])
    if rem:
        pltpu.sync_copy(agg_sh.at[pl.ds(r0 + (RPT // BLK) * BLK, rem), :],
                        out_hbm.at[pl.ds(c * NP + r0 + (RPT // BLK) * BLK, rem), :])


def _agg_call(hs, src2d, dst2d):
    fn = pl.kernel(
        _agg_body,
        out_type=jax.ShapeDtypeStruct((NC * NP, D), jnp.float32),
        mesh=_mesh(),
        scratch_types=[
            pltpu.VMEM((K0, BLK), jnp.int32),
            pltpu.VMEM((K0, BLK), jnp.int32),
            pltpu.VMEM((BLK, D), jnp.float32),
            pltpu.VMEM((BLK, D), jnp.float32),
            pltpu.VMEM_SHARED((NP, D), jnp.float32),
            pltpu.SemaphoreType.DMA,
            pltpu.SemaphoreType.DMA,
        ],
    )
    return fn(hs, src2d, dst2d)


def _h_body(x_ref, w_ref, h_ref):
    h_ref[...] = jnp.dot(x_ref[...], w_ref[...],
                         preferred_element_type=jnp.float32)


def _h_call(x, w_conv2):
    return pl.pallas_call(
        _h_body,
        grid=(NB,),
        in_specs=[
            pl.BlockSpec((BLK, D), lambda i: (i, 0)),
            pl.BlockSpec((D, D), lambda i: (0, 0)),
        ],
        out_specs=pl.BlockSpec((BLK, D), lambda i: (i, 0)),
        out_shape=jax.ShapeDtypeStruct((NP, D), jnp.float32),
    )(x, w_conv2)


def _hs_body(h_ref, deg_ref, hs_ref, dinv_ref):
    i = pl.program_id(0)
    deg_row = jnp.sum(deg_ref[...], axis=0, keepdims=True) + 1.0
    eye = (lax.broadcasted_iota(jnp.int32, (BLK, BLK), 0)
           == lax.broadcasted_iota(jnp.int32, (BLK, BLK), 1)
           ).astype(jnp.float32)
    dn = (((1,), (1,)), ((), ()))
    deg_col = lax.dot_general(eye, deg_row, dn,
                              preferred_element_type=jnp.float32)
    dinv = lax.rsqrt(deg_col)
    rows = i * BLK + lax.broadcasted_iota(jnp.int32, (BLK, 1), 0)
    valid = rows < N
    dinv = jnp.where(valid, dinv, 1.0)
    hs = jnp.where(valid, h_ref[...] * dinv, 0.0)
    hs_ref[...] = hs
    dinv_ref[...] = dinv


def _hs_call(h, deg_all):
    return pl.pallas_call(
        _hs_body,
        grid=(NB,),
        in_specs=[
            pl.BlockSpec((BLK, D), lambda i: (i, 0)),
            pl.BlockSpec((NW, BLK), lambda i: (0, i)),
        ],
        out_specs=[
            pl.BlockSpec((BLK, D), lambda i: (i, 0)),
            pl.BlockSpec((BLK, 1), lambda i: (i, 0)),
        ],
        out_shape=[
            jax.ShapeDtypeStruct((NP, D), jnp.float32),
            jax.ShapeDtypeStruct((NP, 1), jnp.float32),
        ],
    )(h, deg_all)


def _neq_body(tgt_ref, agga_ref, aggb_ref, hs_ref, dinv_ref, bc_ref, wl_ref,
              bl_ref, wm_ref, bm_ref, wo_ref, bo_ref, q_ref, ne_s, gs_s):
    i = pl.program_id(0)
    blk = i % NB

    @pl.when(i == 0)
    def _():
        gs_s[...] = jnp.zeros_like(gs_s)

    @pl.when(i < NB)
    def _():
        agg = agga_ref[...] + aggb_ref[...] + hs_ref[...]
        conv = jnp.maximum(agg * dinv_ref[...] + bc_ref[...], 0.0)
        ne = jnp.dot(conv, wl_ref[...], preferred_element_type=jnp.float32)
        ne = jnp.maximum(ne + bl_ref[...], 0.0)
        rows = blk * BLK + lax.broadcasted_iota(jnp.int32, (BLK, 1), 0)
        ne = jnp.where(rows < N, ne, 0.0)
        ne_s[pl.ds(blk * BLK, BLK), :] = ne
        gs_s[...] += jnp.sum(ne, axis=0, keepdims=True)

    @pl.when(i >= NB)
    def _():
        g = gs_s[...] * (1.0 / N)
        wm = wm_ref[...]
        bias = jnp.dot(g, wm[D:, :], preferred_element_type=jnp.float32)
        bias = bias + bm_ref[...]
        t = ne_s[pl.ds(tgt_ref[0], 1), :]
        dn = (((1,), (1,)), ((), ()))
        v = lax.dot_general(wo_ref[...], t, dn,
                            preferred_element_type=jnp.float32)
        cc = lax.dot_general(bo_ref[...], t, dn,
                             preferred_element_type=jnp.float32)
        ne = ne_s[pl.ds(blk * BLK, BLK), :]
        hid = jnp.dot(ne, wm[:D, :], preferred_element_type=jnp.float32)
        hid = jnp.maximum(hid + bias, 0.0)
        q_ref[...] = jnp.dot(hid, v, preferred_element_type=jnp.float32) + cc


def _neq_call(target_node, agg2, hs, dinv, b_conv2, w_lin1, b_lin1, w_mlp,
              b_mlp, w_out, b_out):
    z = lambda i: (0, 0)
    blk_d = lambda i: (i % NB, 0)
    return pl.pallas_call(
        _neq_body,
        grid=(2 * NB,),
        in_specs=[
            pl.BlockSpec(memory_space=pltpu.SMEM),
            pl.BlockSpec((BLK, D), blk_d),
            pl.BlockSpec((BLK, D), lambda i: (i % NB + NB, 0)),
            pl.BlockSpec((BLK, D), blk_d),
            pl.BlockSpec((BLK, 1), blk_d),
            pl.BlockSpec((1, D), z),
            pl.BlockSpec((D, D), z),
            pl.BlockSpec((1, D), z),
            pl.BlockSpec((2 * D, H), z),
            pl.BlockSpec((1, H), z),
            pl.BlockSpec((H, D), z),
            pl.BlockSpec((1, D), z),
        ],
        out_specs=pl.BlockSpec((BLK, 1),
                               lambda i: (jnp.where(i < NB, 0, i - NB), 0)),
        out_shape=jax.ShapeDtypeStruct((N, 1), jnp.float32),
        scratch_shapes=[
            pltpu.VMEM((NP, D), jnp.float32),
            pltpu.VMEM((1, D), jnp.float32),
        ],
    )(target_node, agg2, agg2, hs, dinv, b_conv2.reshape(1, D), w_lin1,
      b_lin1.reshape(1, D), w_mlp, b_mlp.reshape(1, H), w_out,
      b_out.reshape(1, D))


def kernel(x, edge_index, target_node, W_conv2, b_conv2, W_lin1, b_lin1,
           W_mlp, b_mlp, W_out, b_out):
    src2d = jnp.pad(edge_index[0], (0, EP - E),
                    constant_values=N).reshape(ER, BLK)
    dst2d = jnp.pad(edge_index[1], (0, EP - E),
                    constant_values=N).reshape(ER, BLK)
    tgt = jnp.asarray(target_node, jnp.int32).reshape(1)

    degr = _deg_call(dst2d)
    deg_all = degr.reshape(NW, DP)
    h = _h_call(x, W_conv2)
    hs, dinv = _hs_call(h, deg_all)
    agg2 = _agg_call(hs, src2d, dst2d)
    return _neq_call(tgt, agg2, hs, dinv, b_conv2, W_lin1, b_lin1,
                     W_mlp, b_mlp, W_out, b_out)


# speedup vs baseline: 1.6888x; 1.4321x over previous
import functools

import jax
import jax.numpy as jnp
from jax import lax
from jax.experimental import pallas as pl
from jax.experimental.pallas import tpu as pltpu
from jax.experimental.pallas import tpu_sc as plsc

N = 10000
E = 160000
D = 128
H = 128

NC = 2
NS = 16
NW = NC * NS

BLK = 128
NP = 10112
RPT = NP // NS
NH = 10240
BR = 1024
NBB = NH // BR
EP = 163840
ER = EP // BLK
ERW = ER // NW


def _mesh():
    return plsc.VectorSubcoreMesh(
        core_axis_name="c", subcore_axis_name="s", num_cores=NC, num_subcores=NS)


DP = 10240


def _deg_body(dst_hbm, out_hbm, dst_v, degp):
    c = lax.axis_index("c")
    s = lax.axis_index("s")
    wid = s * NC + c
    zer = jnp.zeros((16,), jnp.float32)
    one = jnp.full((16,), 1.0, jnp.float32)

    def zfill(i, carry):
        degp[pl.ds(i * 16, 16)] = zer
        return carry
    lax.fori_loop(0, DP // 16, zfill, 0)
    pltpu.sync_copy(dst_hbm.at[pl.ds(wid * ERW, ERW), :], dst_v)

    def body(i, carry):
        j = i // 8
        k = (i % 8) * 16
        d = dst_v[j, pl.ds(k, 16)]
        plsc.addupdate_scatter(degp, [d], one)
        return carry
    lax.fori_loop(0, ERW * 8, body, 0)
    pltpu.sync_copy(degp, out_hbm.at[pl.ds(wid * DP, DP)])


def _deg_call(dst2d):
    fn = pl.kernel(
        _deg_body,
        out_type=jax.ShapeDtypeStruct((NW * DP,), jnp.float32),
        mesh=_mesh(),
        compiler_params=pltpu.CompilerParams(needs_layout_passes=False),
        scratch_types=[
            pltpu.VMEM((ERW, BLK), jnp.int32),
            pltpu.VMEM((DP,), jnp.float32),
        ],
    )
    return fn(dst2d)


K0 = 64
K1 = ER // NS - K0


def _agg_body(hs_hbm, src_hbm, dst_hbm, out_hbm, src_v, dst_r, buf_a, buf_b,
              agg_sh, sem, sem_i):
    c = lax.axis_index("c")
    s = lax.axis_index("s")
    zer = jnp.zeros((16,), jnp.float32)

    def zfill(i, carry):
        r = i // 8
        k = (i % 8) * 16
        buf_a[r, pl.ds(k, 16)] = zer
        return carry
    lax.fori_loop(0, BLK * 8, zfill, 0)

    r0 = s * RPT
    for k in range(RPT // BLK):
        pltpu.sync_copy(buf_a, agg_sh.at[pl.ds(r0 + k * BLK, BLK), :])
    rem = RPT % BLK
    if rem:
        pltpu.sync_copy(buf_a.at[pl.ds(0, rem), :],
                        agg_sh.at[pl.ds(r0 + (RPT // BLK) * BLK, rem), :])

    def run_edges(K, base):
        pltpu.sync_copy(src_hbm.at[pl.ds(base, K), :], src_v.at[pl.ds(0, K), :])
        pltpu.sync_copy(dst_hbm.at[pl.ds(base, K), :], dst_r.at[pl.ds(0, K), :])
        pltpu.async_copy(hs_hbm.at[src_v.at[0]], buf_a, sem)

        def body(t, carry):
            j0 = 2 * t
            j1 = j0 + 1
            pltpu.make_async_copy(hs_hbm.at[src_v.at[j0]], buf_a, sem).wait()
            pltpu.async_copy(hs_hbm.at[src_v.at[j1]], buf_b, sem)
            pltpu.sync_copy(buf_a, agg_sh.at[dst_r.at[j0]], add=True)
            pltpu.make_async_copy(hs_hbm.at[src_v.at[j1]], buf_b, sem).wait()

            @pl.when(j1 + 1 < K)
            def _():
                pltpu.async_copy(hs_hbm.at[src_v.at[j1 + 1]], buf_a, sem)
            pltpu.sync_copy(buf_b, agg_sh.at[dst_r.at[j1]], add=True)
            return carry
        lax.fori_loop(0, K // 2, body, 0)

    @pl.when(c == 0)
    def _():
        run_edges(K0, s * K0)

    if K1 > 0:
        @pl.when(c == 1)
        def _():
            run_edges(K1, NS * K0 + s * K1)
    plsc.subcore_barrier()
    for k in range(RPT // BLK):
        pltpu.sync_copy(agg_sh.at[pl.ds(r0 + k * BLK, BLK), :],
                        out_hbm.at[pl.ds(c * NH + r0 + k * BLK, BLK), :])
    if rem:
        pltpu.sync_copy(agg_sh.at[pl.ds(r0 + (RPT // BLK) * BLK, rem), :],
                        out_hbm.at[pl.ds(c * NH + r0 + (RPT // BLK) * BLK, rem), :])


def _agg_call(hs, src2d, dst2d):
    fn = pl.kernel(
        _agg_body,
        out_type=jax.ShapeDtypeStruct((NC * NH, D), jnp.float32),
        mesh=_mesh(),
        scratch_types=[
            pltpu.VMEM((K0, BLK), jnp.int32),
            pltpu.VMEM((K0, BLK), jnp.int32),
            pltpu.VMEM((BLK, D), jnp.float32),
            pltpu.VMEM((BLK, D), jnp.float32),
            pltpu.VMEM_SHARED((NP, D), jnp.float32),
            pltpu.SemaphoreType.DMA,
            pltpu.SemaphoreType.DMA,
        ],
    )
    return fn(hs, src2d, dst2d)


def _h_body(x_ref, w_ref, h_ref):
    h_ref[...] = jnp.dot(x_ref[...], w_ref[...],
                         preferred_element_type=jnp.float32)


def _h_call(x, w_conv2):
    return pl.pallas_call(
        _h_body,
        grid=(NBB,),
        in_specs=[
            pl.BlockSpec((BR, D), lambda i: (i, 0)),
            pl.BlockSpec((D, D), lambda i: (0, 0)),
        ],
        out_specs=pl.BlockSpec((BR, D), lambda i: (i, 0)),
        out_shape=jax.ShapeDtypeStruct((NH, D), jnp.float32),
    )(x, w_conv2)


def _hs_body(h_ref, deg_ref, hs_ref, dinv_ref):
    i = pl.program_id(0)
    deg_row = jnp.sum(deg_ref[...], axis=0, keepdims=True) + 1.0
    eye = (lax.broadcasted_iota(jnp.int32, (BLK, BLK), 0)
           == lax.broadcasted_iota(jnp.int32, (BLK, BLK), 1)
           ).astype(jnp.float32)
    dn = (((1,), (1,)), ((), ()))
    cols = [lax.dot_general(eye, deg_row[:, p * BLK:(p + 1) * BLK], dn,
                            preferred_element_type=jnp.float32)
            for p in range(BR // BLK)]
    deg_col = jnp.concatenate(cols, axis=0)
    dinv = lax.rsqrt(deg_col)
    rows = i * BR + lax.broadcasted_iota(jnp.int32, (BR, 1), 0)
    valid = rows < N
    dinv = jnp.where(valid, dinv, 1.0)
    hs = jnp.where(valid, h_ref[...] * dinv, 0.0)
    hs_ref[...] = hs
    dinv_ref[...] = dinv


def _hs_call(h, deg_all):
    return pl.pallas_call(
        _hs_body,
        grid=(NBB,),
        in_specs=[
            pl.BlockSpec((BR, D), lambda i: (i, 0)),
            pl.BlockSpec((NW, BR), lambda i: (0, i)),
        ],
        out_specs=[
            pl.BlockSpec((BR, D), lambda i: (i, 0)),
            pl.BlockSpec((BR, 1), lambda i: (i, 0)),
        ],
        out_shape=[
            jax.ShapeDtypeStruct((NH, D), jnp.float32),
            jax.ShapeDtypeStruct((NH, 1), jnp.float32),
        ],
    )(h, deg_all)


def _neq_body(tgt_ref, agga_ref, aggb_ref, hs_ref, dinv_ref, bc_ref, wl_ref,
              bl_ref, wm_ref, bm_ref, wo_ref, bo_ref, q_ref, ne_s, gs_s):
    i = pl.program_id(0)
    blk = i % NBB

    @pl.when(i == 0)
    def _():
        gs_s[...] = jnp.zeros_like(gs_s)

    @pl.when(i < NBB)
    def _():
        agg = agga_ref[...] + aggb_ref[...] + hs_ref[...]
        conv = jnp.maximum(agg * dinv_ref[...] + bc_ref[...], 0.0)
        ne = jnp.dot(conv, wl_ref[...], preferred_element_type=jnp.float32)
        ne = jnp.maximum(ne + bl_ref[...], 0.0)
        rows = blk * BR + lax.broadcasted_iota(jnp.int32, (BR, 1), 0)
        ne = jnp.where(rows < N, ne, 0.0)
        ne_s[pl.ds(blk * BR, BR), :] = ne
        gs_s[...] += jnp.sum(ne, axis=0, keepdims=True)

    @pl.when(i >= NBB)
    def _():
        g = gs_s[...] * (1.0 / N)
        wm = wm_ref[...]
        bias = jnp.dot(g, wm[D:, :], preferred_element_type=jnp.float32)
        bias = bias + bm_ref[...]
        t = ne_s[pl.ds(tgt_ref[0], 1), :]
        dn = (((1,), (1,)), ((), ()))
        v = lax.dot_general(wo_ref[...], t, dn,
                            preferred_element_type=jnp.float32)
        cc = lax.dot_general(bo_ref[...], t, dn,
                             preferred_element_type=jnp.float32)
        ne = ne_s[pl.ds(blk * BR, BR), :]
        hid = jnp.dot(ne, wm[:D, :], preferred_element_type=jnp.float32)
        hid = jnp.maximum(hid + bias, 0.0)
        q_ref[...] = jnp.dot(hid, v, preferred_element_type=jnp.float32) + cc


def _neq_call(target_node, agg2, hs, dinv, b_conv2, w_lin1, b_lin1, w_mlp,
              b_mlp, w_out, b_out):
    z = lambda i: (0, 0)
    blk_d = lambda i: (i % NBB, 0)
    return pl.pallas_call(
        _neq_body,
        grid=(2 * NBB,),
        in_specs=[
            pl.BlockSpec(memory_space=pltpu.SMEM),
            pl.BlockSpec((BR, D), blk_d),
            pl.BlockSpec((BR, D), lambda i: (i % NBB + NBB, 0)),
            pl.BlockSpec((BR, D), blk_d),
            pl.BlockSpec((BR, 1), blk_d),
            pl.BlockSpec((1, D), z),
            pl.BlockSpec((D, D), z),
            pl.BlockSpec((1, D), z),
            pl.BlockSpec((2 * D, H), z),
            pl.BlockSpec((1, H), z),
            pl.BlockSpec((H, D), z),
            pl.BlockSpec((1, D), z),
        ],
        out_specs=pl.BlockSpec((BR, 1),
                               lambda i: (jnp.where(i < NBB, 0, i - NBB), 0)),
        out_shape=jax.ShapeDtypeStruct((N, 1), jnp.float32),
        scratch_shapes=[
            pltpu.VMEM((NH, D), jnp.float32),
            pltpu.VMEM((1, D), jnp.float32),
        ],
    )(target_node, agg2, agg2, hs, dinv, b_conv2.reshape(1, D), w_lin1,
      b_lin1.reshape(1, D), w_mlp, b_mlp.reshape(1, H), w_out,
      b_out.reshape(1, D))


def kernel(x, edge_index, target_node, W_conv2, b_conv2, W_lin1, b_lin1,
           W_mlp, b_mlp, W_out, b_out):
    src2d = jnp.pad(edge_index[0], (0, EP - E),
                    constant_values=N).reshape(ER, BLK)
    dst2d = jnp.pad(edge_index[1], (0, EP - E),
                    constant_values=N).reshape(ER, BLK)
    tgt = jnp.asarray(target_node, jnp.int32).reshape(1)

    degr = _deg_call(dst2d)
    deg_all = degr.reshape(NW, DP)
    h = _h_call(x, W_conv2)
    hs, dinv = _hs_call(h, deg_all)
    agg2 = _agg_call(hs, src2d, dst2d)
    return _neq_call(tgt, agg2, hs, dinv, b_conv2, W_lin1, b_lin1,
                     W_mlp, b_mlp, W_out, b_out)


# speedup vs baseline: 1.6903x; 1.0009x over previous
import functools

import jax
import jax.numpy as jnp
from jax import lax
from jax.experimental import pallas as pl
from jax.experimental.pallas import tpu as pltpu
from jax.experimental.pallas import tpu_sc as plsc

N = 10000
E = 160000
D = 128
H = 128

NC = 2
NS = 16
NW = NC * NS

BLK = 128
NP = 10112
RPT = NP // NS
NH = 10240
BR = 1024
NBB = NH // BR
EP = 163840
ER = EP // BLK
ERW = ER // NW


def _mesh():
    return plsc.VectorSubcoreMesh(
        core_axis_name="c", subcore_axis_name="s", num_cores=NC, num_subcores=NS)


DP = 10240


def _deg_body(dst_hbm, out_hbm, dst_v, degp):
    c = lax.axis_index("c")
    s = lax.axis_index("s")
    wid = s * NC + c
    zer = jnp.zeros((16,), jnp.float32)
    one = jnp.full((16,), 1.0, jnp.float32)

    def zfill(i, carry):
        degp[pl.ds(i * 16, 16)] = zer
        return carry
    lax.fori_loop(0, DP // 16, zfill, 0)
    pltpu.sync_copy(dst_hbm.at[pl.ds(wid * ERW, ERW), :], dst_v)

    def body(i, carry):
        j = i // 8
        k = (i % 8) * 16
        d = dst_v[j, pl.ds(k, 16)]
        plsc.addupdate_scatter(degp, [d], one)
        return carry
    lax.fori_loop(0, ERW * 8, body, 0)
    pltpu.sync_copy(degp, out_hbm.at[pl.ds(wid * DP, DP)])


def _deg_call(dst2d):
    fn = pl.kernel(
        _deg_body,
        out_type=jax.ShapeDtypeStruct((NW * DP,), jnp.float32),
        mesh=_mesh(),
        compiler_params=pltpu.CompilerParams(needs_layout_passes=False),
        scratch_types=[
            pltpu.VMEM((ERW, BLK), jnp.int32),
            pltpu.VMEM((DP,), jnp.float32),
        ],
    )
    return fn(dst2d)


K0 = 64
K1 = ER // NS - K0


def _agg_body(hs_hbm, hs2_hbm, src_hbm, dst_hbm, out_hbm, src_v, dst_r,
              buf_a, buf_b, agg_sh, sem, sem_i):
    c = lax.axis_index("c")
    s = lax.axis_index("s")
    zer = jnp.zeros((16,), jnp.float32)

    def zfill(i, carry):
        r = i // 8
        k = (i % 8) * 16
        buf_a[r, pl.ds(k, 16)] = zer
        return carry
    lax.fori_loop(0, BLK * 8, zfill, 0)

    r0 = s * RPT
    for k in range(RPT // BLK):
        pltpu.sync_copy(buf_a, agg_sh.at[pl.ds(r0 + k * BLK, BLK), :])
    rem = RPT % BLK
    if rem:
        pltpu.sync_copy(buf_a.at[pl.ds(0, rem), :],
                        agg_sh.at[pl.ds(r0 + (RPT // BLK) * BLK, rem), :])

    def run_edges(h_hbm, K, base):
        pltpu.sync_copy(src_hbm.at[pl.ds(base, K), :], src_v.at[pl.ds(0, K), :])
        pltpu.sync_copy(dst_hbm.at[pl.ds(base, K), :], dst_r.at[pl.ds(0, K), :])
        pltpu.async_copy(h_hbm.at[src_v.at[0]], buf_a, sem)

        def body(t, carry):
            j0 = 2 * t
            j1 = j0 + 1
            pltpu.make_async_copy(h_hbm.at[src_v.at[j0]], buf_a, sem).wait()
            pltpu.async_copy(h_hbm.at[src_v.at[j1]], buf_b, sem)
            pltpu.sync_copy(buf_a, agg_sh.at[dst_r.at[j0]], add=True)
            pltpu.make_async_copy(h_hbm.at[src_v.at[j1]], buf_b, sem).wait()

            @pl.when(j1 + 1 < K)
            def _():
                pltpu.async_copy(h_hbm.at[src_v.at[j1 + 1]], buf_a, sem)
            pltpu.sync_copy(buf_b, agg_sh.at[dst_r.at[j1]], add=True)
            return carry
        lax.fori_loop(0, K // 2, body, 0)

    @pl.when(c == 0)
    def _():
        run_edges(hs_hbm, K0, s * K0)

    if K1 > 0:
        @pl.when(c == 1)
        def _():
            run_edges(hs2_hbm, K1, NS * K0 + s * K1)
    plsc.subcore_barrier()
    for k in range(RPT // BLK):
        pltpu.sync_copy(agg_sh.at[pl.ds(r0 + k * BLK, BLK), :],
                        out_hbm.at[pl.ds(c * NH + r0 + k * BLK, BLK), :])
    if rem:
        pltpu.sync_copy(agg_sh.at[pl.ds(r0 + (RPT // BLK) * BLK, rem), :],
                        out_hbm.at[pl.ds(c * NH + r0 + (RPT // BLK) * BLK, rem), :])


def _agg_call(hs, hs2, src2d, dst2d):
    fn = pl.kernel(
        _agg_body,
        out_type=jax.ShapeDtypeStruct((NC * NH, D), jnp.float32),
        mesh=_mesh(),
        scratch_types=[
            pltpu.VMEM((K0, BLK), jnp.int32),
            pltpu.VMEM((K0, BLK), jnp.int32),
            pltpu.VMEM((BLK, D), jnp.float32),
            pltpu.VMEM((BLK, D), jnp.float32),
            pltpu.VMEM_SHARED((NP, D), jnp.float32),
            pltpu.SemaphoreType.DMA,
            pltpu.SemaphoreType.DMA,
        ],
    )
    return fn(hs, hs2, src2d, dst2d)


def _h_body(x_ref, w_ref, h_ref):
    h_ref[...] = jnp.dot(x_ref[...], w_ref[...],
                         preferred_element_type=jnp.float32)


def _h_call(x, w_conv2):
    return pl.pallas_call(
        _h_body,
        grid=(NBB,),
        in_specs=[
            pl.BlockSpec((BR, D), lambda i: (i, 0)),
            pl.BlockSpec((D, D), lambda i: (0, 0)),
        ],
        out_specs=pl.BlockSpec((BR, D), lambda i: (i, 0)),
        out_shape=jax.ShapeDtypeStruct((NH, D), jnp.float32),
    )(x, w_conv2)


def _hs_body(h_ref, deg_ref, hs_ref, hs2_ref, dinv_ref):
    i = pl.program_id(0)
    deg_row = jnp.sum(deg_ref[...], axis=0, keepdims=True) + 1.0
    eye = (lax.broadcasted_iota(jnp.int32, (BLK, BLK), 0)
           == lax.broadcasted_iota(jnp.int32, (BLK, BLK), 1)
           ).astype(jnp.float32)
    dn = (((1,), (1,)), ((), ()))
    cols = [lax.dot_general(eye, deg_row[:, p * BLK:(p + 1) * BLK], dn,
                            preferred_element_type=jnp.float32)
            for p in range(BR // BLK)]
    deg_col = jnp.concatenate(cols, axis=0)
    dinv = lax.rsqrt(deg_col)
    rows = i * BR + lax.broadcasted_iota(jnp.int32, (BR, 1), 0)
    valid = rows < N
    dinv = jnp.where(valid, dinv, 1.0)
    hs = jnp.where(valid, h_ref[...] * dinv, 0.0)
    hs_ref[...] = hs
    hs2_ref[...] = hs
    dinv_ref[...] = dinv


def _hs_call(h, deg_all):
    return pl.pallas_call(
        _hs_body,
        grid=(NBB,),
        in_specs=[
            pl.BlockSpec((BR, D), lambda i: (i, 0)),
            pl.BlockSpec((NW, BR), lambda i: (0, i)),
        ],
        out_specs=[
            pl.BlockSpec((BR, D), lambda i: (i, 0)),
            pl.BlockSpec((BR, D), lambda i: (i, 0)),
            pl.BlockSpec((BR, 1), lambda i: (i, 0)),
        ],
        out_shape=[
            jax.ShapeDtypeStruct((NH, D), jnp.float32),
            jax.ShapeDtypeStruct((NH, D), jnp.float32),
            jax.ShapeDtypeStruct((NH, 1), jnp.float32),
        ],
    )(h, deg_all)


def _neq_body(tgt_ref, agga_ref, aggb_ref, hs_ref, dinv_ref, bc_ref, wl_ref,
              bl_ref, wm_ref, bm_ref, wo_ref, bo_ref, q_ref, ne_s, gs_s):
    i = pl.program_id(0)
    blk = i % NBB

    @pl.when(i == 0)
    def _():
        gs_s[...] = jnp.zeros_like(gs_s)

    @pl.when(i < NBB)
    def _():
        agg = agga_ref[...] + aggb_ref[...] + hs_ref[...]
        conv = jnp.maximum(agg * dinv_ref[...] + bc_ref[...], 0.0)
        ne = jnp.dot(conv, wl_ref[...], preferred_element_type=jnp.float32)
        ne = jnp.maximum(ne + bl_ref[...], 0.0)
        rows = blk * BR + lax.broadcasted_iota(jnp.int32, (BR, 1), 0)
        ne = jnp.where(rows < N, ne, 0.0)
        ne_s[pl.ds(blk * BR, BR), :] = ne
        gs_s[...] += jnp.sum(ne, axis=0, keepdims=True)

    @pl.when(i >= NBB)
    def _():
        g = gs_s[...] * (1.0 / N)
        wm = wm_ref[...]
        bias = jnp.dot(g, wm[D:, :], preferred_element_type=jnp.float32)
        bias = bias + bm_ref[...]
        t = ne_s[pl.ds(tgt_ref[0], 1), :]
        dn = (((1,), (1,)), ((), ()))
        v = lax.dot_general(wo_ref[...], t, dn,
                            preferred_element_type=jnp.float32)
        cc = lax.dot_general(bo_ref[...], t, dn,
                             preferred_element_type=jnp.float32)
        ne = ne_s[pl.ds(blk * BR, BR), :]
        hid = jnp.dot(ne, wm[:D, :], preferred_element_type=jnp.float32)
        hid = jnp.maximum(hid + bias, 0.0)
        q_ref[...] = jnp.dot(hid, v, preferred_element_type=jnp.float32) + cc


def _neq_call(target_node, agg2, hs, dinv, b_conv2, w_lin1, b_lin1, w_mlp,
              b_mlp, w_out, b_out):
    z = lambda i: (0, 0)
    blk_d = lambda i: (i % NBB, 0)
    return pl.pallas_call(
        _neq_body,
        grid=(2 * NBB,),
        in_specs=[
            pl.BlockSpec(memory_space=pltpu.SMEM),
            pl.BlockSpec((BR, D), blk_d),
            pl.BlockSpec((BR, D), lambda i: (i % NBB + NBB, 0)),
            pl.BlockSpec((BR, D), blk_d),
            pl.BlockSpec((BR, 1), blk_d),
            pl.BlockSpec((1, D), z),
            pl.BlockSpec((D, D), z),
            pl.BlockSpec((1, D), z),
            pl.BlockSpec((2 * D, H), z),
            pl.BlockSpec((1, H), z),
            pl.BlockSpec((H, D), z),
            pl.BlockSpec((1, D), z),
        ],
        out_specs=pl.BlockSpec((BR, 1),
                               lambda i: (jnp.where(i < NBB, 0, i - NBB), 0)),
        out_shape=jax.ShapeDtypeStruct((N, 1), jnp.float32),
        scratch_shapes=[
            pltpu.VMEM((NH, D), jnp.float32),
            pltpu.VMEM((1, D), jnp.float32),
        ],
    )(target_node, agg2, agg2, hs, dinv, b_conv2.reshape(1, D), w_lin1,
      b_lin1.reshape(1, D), w_mlp, b_mlp.reshape(1, H), w_out,
      b_out.reshape(1, D))


def kernel(x, edge_index, target_node, W_conv2, b_conv2, W_lin1, b_lin1,
           W_mlp, b_mlp, W_out, b_out):
    src2d = jnp.pad(edge_index[0], (0, EP - E),
                    constant_values=N).reshape(ER, BLK)
    dst2d = jnp.pad(edge_index[1], (0, EP - E),
                    constant_values=N).reshape(ER, BLK)
    tgt = jnp.asarray(target_node, jnp.int32).reshape(1)

    degr = _deg_call(dst2d)
    deg_all = degr.reshape(NW, DP)
    h = _h_call(x, W_conv2)
    hs, hs2, dinv = _hs_call(h, deg_all)
    agg2 = _agg_call(hs, hs2, src2d, dst2d)
    return _neq_call(tgt, agg2, hs, dinv, b_conv2, W_lin1, b_lin1,
                     W_mlp, b_mlp, W_out, b_out)
